# baseline (speedup 1.0000x reference)
import functools

import jax
import jax.numpy as jnp
from jax import lax
from jax.experimental import pallas as pl
from jax.experimental.pallas import tpu as pltpu
from jax.experimental.pallas import tpu_sc as plsc

_THRESHOLD = 0.9952
_LANES = 16


def kernel(input, filt):
    B, T = input.shape
    F = filt.shape[0]
    n_steps = T - F
    n_blocks = n_steps // F

    info = plsc.get_sparse_core_info()
    nc, ns = info.num_cores, info.num_subcores
    n_workers = nc * ns
    rows_per_worker = B // n_workers
    chunk = rows_per_worker * T
    nv = F // rows_per_worker

    mesh = plsc.VectorSubcoreMesh(core_axis_name="c", subcore_axis_name="s")

    @functools.partial(
        pl.kernel,
        mesh=mesh,
        compiler_params=pltpu.CompilerParams(needs_layout_passes=False),
        out_type=jax.ShapeDtypeStruct((B * T,), jnp.float32),
        scratch_types=[
            pltpu.VMEM((chunk,), jnp.float32),
            pltpu.VMEM((chunk,), jnp.float32),
            pltpu.VMEM((F,), jnp.float32),
            pltpu.VMEM((F * nv * _LANES,), jnp.float32),
        ],
    )
    def bsa(x_hbm, filt_hbm, out_hbm, x_v, out_v, filt_v, ftab_v):
        wid = lax.axis_index("s") * nc + lax.axis_index("c")
        base = wid * chunk
        pltpu.sync_copy(x_hbm.at[pl.ds(base, chunk)], x_v)
        pltpu.sync_copy(filt_hbm, filt_v)

        lane = lax.iota(jnp.int32, _LANES)
        row4 = (lane >> 2) * T
        pos4 = lane & 3
        thr = jnp.float32(_THRESHOLD)
        one_v = jnp.ones((_LANES,), jnp.float32)
        zero_v = jnp.zeros((_LANES,), jnp.float32)
        lane0 = pos4 == 0
        posmask = [pos4 == i for i in range(rows_per_worker)]
        xor2 = jax.lax.bitwise_xor(lane, 2)
        xor1 = jax.lax.bitwise_xor(lane, 1)

        def shuf(vv, idxv):
            return jnp.take_along_axis(vv, idxv, axis=0,
                                       mode="promise_in_bounds")

        def tree(a):
            p1a = a[0] + a[2]
            p1b = a[1] + a[3]
            p2 = p1a + p1b
            p3 = p2 + shuf(p2, xor2)
            return p3 + shuf(p3, xor1)

        for r in range(F):
            for i in range(nv):
                idx = (pos4 + ((4 * i - r) % F)) & (F - 1)
                ftab_v[pl.ds((r * nv + i) * _LANES, _LANES)] = (
                    plsc.load_gather(filt_v, [idx]))

        def spec(w, taps):
            u = [w[i] - taps[i] for i in range(nv)]
            uu = [u[i] - taps[i] for i in range(nv)]
            a_u = jnp.abs(tree(u))
            a_uu = jnp.abs(tree(uu))
            b_u = a_u * thr
            b_w = jnp.abs(tree(w)) * thr
            return u, a_u, a_uu, b_u, b_w

        w = [plsc.load_gather(x_v, [row4 + pos4 + 4 * i]) for i in range(nv)]
        taps0 = [ftab_v[pl.ds(i * _LANES, _LANES)] for i in range(nv)]
        u, a_u, a_uu, b_u, b_w = spec(w, taps0)
        m0 = jnp.zeros((_LANES,), jnp.bool_)

        def block(jb, carry):
            w = list(carry[:nv])
            u = list(carry[nv:2 * nv])
            a_u, a_uu, b_u, b_w, m_prev = carry[2 * nv:]
            bvec0 = row4 + jb * F
            for k in range(F):
                idx_out = bvec0 + k
                e1 = jnp.where(m_prev, a_uu, a_u)
                e2 = jnp.where(m_prev, b_u, b_w)
                m = e1 <= e2
                spike = jnp.where(m, one_v, zero_v)
                plsc.store_scatter(out_v, [idx_out], spike, mask=lane0)
                xnew = plsc.load_gather(x_v, [idx_out + F])
                w = [jnp.where(m_prev, u[i], w[i]) for i in range(nv)]
                ke, le = k // 4, k % 4
                w[ke] = jnp.where(posmask[le], xnew, w[ke])
                rn = (k + 1) % F
                taps = [ftab_v[pl.ds((rn * nv + i) * _LANES, _LANES)]
                        for i in range(nv)]
                u, a_u, a_uu, b_u, b_w = spec(w, taps)
                m_prev = m
            return (*w, *u, a_u, a_uu, b_u, b_w, m_prev)

        lax.fori_loop(0, n_blocks, block, (*w, *u, a_u, a_uu, b_u, b_w, m0))

        for j in range(n_steps, T):
            plsc.store_scatter(out_v, [row4 + j], zero_v, mask=lane0)

        pltpu.sync_copy(out_v, out_hbm.at[pl.ds(base, chunk)])

    out_flat = bsa(input.reshape(B * T), filt)
    return out_flat.reshape(B, T)


# speedup vs baseline: 1.0170x; 1.0170x over previous
import functools

import jax
import jax.numpy as jnp
from jax import lax
from jax.experimental import pallas as pl
from jax.experimental.pallas import tpu as pltpu
from jax.experimental.pallas import tpu_sc as plsc

_THRESHOLD = 0.9952
_LANES = 16


def kernel(input, filt):
    B, T = input.shape
    F = filt.shape[0]
    n_steps = T - F
    n_blocks = n_steps // F

    info = plsc.get_sparse_core_info()
    nc, ns = info.num_cores, info.num_subcores
    n_workers = nc * ns
    rows_per_worker = B // n_workers
    chunk = rows_per_worker * T
    nv = F // rows_per_worker

    mesh = plsc.VectorSubcoreMesh(core_axis_name="c", subcore_axis_name="s")

    @functools.partial(
        pl.kernel,
        mesh=mesh,
        compiler_params=pltpu.CompilerParams(needs_layout_passes=False),
        out_type=jax.ShapeDtypeStruct((B * T,), jnp.float32),
        scratch_types=[
            pltpu.VMEM((chunk,), jnp.float32),
            pltpu.VMEM((chunk,), jnp.float32),
            pltpu.VMEM((F,), jnp.float32),
            pltpu.VMEM((F * nv * _LANES,), jnp.float32),
        ],
    )
    def bsa(x_hbm, filt_hbm, out_hbm, x_v, out_v, filt_v, ftab_v):
        wid = lax.axis_index("s") * nc + lax.axis_index("c")
        base = wid * chunk
        pltpu.sync_copy(x_hbm.at[pl.ds(base, chunk)], x_v)
        pltpu.sync_copy(filt_hbm, filt_v)

        lane = lax.iota(jnp.int32, _LANES)
        row4 = (lane >> 2) * T
        pos4 = lane & 3
        thr = jnp.float32(_THRESHOLD)
        one_v = jnp.ones((_LANES,), jnp.float32)
        zero_v = jnp.zeros((_LANES,), jnp.float32)
        lane0 = pos4 == 0
        posmask = [pos4 == i for i in range(rows_per_worker)]
        xor2 = jax.lax.bitwise_xor(lane, 2)
        xor1 = jax.lax.bitwise_xor(lane, 1)

        def shuf(vv, idxv):
            return jnp.take_along_axis(vv, idxv, axis=0,
                                       mode="promise_in_bounds")

        def tree(a):
            p1a = a[0] + a[2]
            p1b = a[1] + a[3]
            p2 = p1a + p1b
            p3 = p2 + shuf(p2, xor2)
            return p3 + shuf(p3, xor1)

        for r in range(F):
            for i in range(nv):
                idx = (pos4 + ((4 * i - r) % F)) & (F - 1)
                ftab_v[pl.ds((r * nv + i) * _LANES, _LANES)] = (
                    plsc.load_gather(filt_v, [idx]))

        v = [plsc.load_gather(x_v, [row4 + pos4 + 4 * i]) for i in range(nv)]
        taps = [ftab_v[pl.ds(i * _LANES, _LANES)] for i in range(nv)]

        def block(jb, carry):
            v = list(carry[:nv])
            taps = list(carry[nv:])
            bvec0 = row4 + jb * F
            for k in range(F):
                idx_out = bvec0 + k
                xnew = plsc.load_gather(x_v, [idx_out + F])
                d1 = [v[i] - taps[i] for i in range(nv)]
                e1 = jnp.abs(tree(d1))
                e2 = jnp.abs(tree(v)) * thr
                m = e1 <= e2
                spike = jnp.where(m, one_v, zero_v)
                plsc.store_scatter(out_v, [idx_out], spike, mask=lane0)
                ke, le = k // 4, k % 4
                v[ke] = jnp.where(posmask[le], xnew, v[ke])
                rn = (k + 1) % F
                taps = [ftab_v[pl.ds((rn * nv + i) * _LANES, _LANES)]
                        for i in range(nv)]
                v = [jnp.where(m, v[i] - taps[i], v[i]) for i in range(nv)]
            return (*v, *taps)

        lax.fori_loop(0, n_blocks, block, (*v, *taps))

        for j in range(n_steps, T):
            plsc.store_scatter(out_v, [row4 + j], zero_v, mask=lane0)

        pltpu.sync_copy(out_v, out_hbm.at[pl.ds(base, chunk)])

    out_flat = bsa(input.reshape(B * T), filt)
    return out_flat.reshape(B, T)


# speedup vs baseline: 1.6184x; 1.5914x over previous
import functools

import jax
import jax.numpy as jnp
from jax import lax
from jax.experimental import pallas as pl
from jax.experimental.pallas import tpu as pltpu
from jax.experimental.pallas import tpu_sc as plsc

_THRESHOLD = 0.9952
_LANES = 16


def kernel(input, filt):
    B, T = input.shape
    F = filt.shape[0]
    n_steps = T - F
    n_blocks = n_steps // F

    info = plsc.get_sparse_core_info()
    nc, ns = info.num_cores, info.num_subcores
    n_workers = nc * ns
    rows_per_worker = B // n_workers
    chunk = rows_per_worker * T
    nv = F // rows_per_worker

    mesh = plsc.VectorSubcoreMesh(core_axis_name="c", subcore_axis_name="s")

    @functools.partial(
        pl.kernel,
        mesh=mesh,
        compiler_params=pltpu.CompilerParams(needs_layout_passes=False),
        out_type=jax.ShapeDtypeStruct((B * T,), jnp.float32),
        scratch_types=[
            pltpu.VMEM((chunk,), jnp.float32),
            pltpu.VMEM((chunk,), jnp.float32),
            pltpu.VMEM((F,), jnp.float32),
            pltpu.VMEM((F * nv * _LANES,), jnp.float32),
        ],
    )
    def bsa(x_hbm, filt_hbm, out_hbm, x_v, out_v, filt_v, ftab_v):
        wid = lax.axis_index("s") * nc + lax.axis_index("c")
        base = wid * chunk
        pltpu.sync_copy(x_hbm.at[pl.ds(base, chunk)], x_v)
        pltpu.sync_copy(filt_hbm, filt_v)

        lane = lax.iota(jnp.int32, _LANES)
        row4 = (lane >> 2) * T
        pos4 = lane & 3
        thr = jnp.float32(_THRESHOLD)
        one_v = jnp.ones((_LANES,), jnp.float32)
        zero_v = jnp.zeros((_LANES,), jnp.float32)
        posmask = [pos4 == i for i in range(rows_per_worker)]
        xor2 = jax.lax.bitwise_xor(lane, 2)
        xor1 = jax.lax.bitwise_xor(lane, 1)
        bitlane = jnp.int32(1) << lane
        zero_i = jnp.zeros((_LANES,), jnp.int32)
        top_i = zero_i + jnp.int32(1 << (F - 1))
        grp_first = [zero_i + jnp.int32(4 * r) for r in range(rows_per_worker)]

        def shuf(vv, idxv):
            return jnp.take_along_axis(vv, idxv, axis=0,
                                       mode="promise_in_bounds")

        def tree(a):
            p1a = a[0] + a[2]
            p1b = a[1] + a[3]
            p2 = p1a + p1b
            p3 = p2 + shuf(p2, xor2)
            return p3 + shuf(p3, xor1)

        for r in range(F):
            for i in range(nv):
                idx = (pos4 + ((4 * i - r) % F)) & (F - 1)
                ftab_v[pl.ds((r * nv + i) * _LANES, _LANES)] = (
                    plsc.load_gather(filt_v, [idx]))

        v = [plsc.load_gather(x_v, [row4 + pos4 + 4 * i]) for i in range(nv)]
        taps = [ftab_v[pl.ds(i * _LANES, _LANES)] for i in range(nv)]

        def block(jb, carry):
            v = list(carry[:nv])
            taps = list(carry[nv:])
            j0 = jb * F
            gvec0 = row4 + pos4 + (j0 + F)
            g = [plsc.load_gather(x_v, [gvec0 + 4 * i]) for i in range(nv)]
            acc = zero_i
            for k in range(F):
                d1 = [v[i] - taps[i] for i in range(nv)]
                e1 = jnp.abs(tree(d1))
                e2 = jnp.abs(tree(v)) * thr
                m = e1 <= e2
                acc = jax.lax.shift_right_logical(acc, 1) | jnp.where(
                    m, top_i, zero_i)
                ke, le = k // 4, k % 4
                v[ke] = jnp.where(posmask[le], g[ke], v[ke])
                rn = (k + 1) % F
                taps = [ftab_v[pl.ds((rn * nv + i) * _LANES, _LANES)]
                        for i in range(nv)]
                v = [jnp.where(m, v[i] - taps[i], v[i]) for i in range(nv)]
            for r in range(rows_per_worker):
                bits = shuf(acc, grp_first[r]) & bitlane
                spikes = jnp.where(bits != zero_i, one_v, zero_v)
                out_v[pl.ds(r * T + j0, F)] = spikes
            return (*v, *taps)

        lax.fori_loop(0, n_blocks, block, (*v, *taps))

        for r in range(rows_per_worker):
            out_v[pl.ds(r * T + n_steps, F)] = zero_v

        pltpu.sync_copy(out_v, out_hbm.at[pl.ds(base, chunk)])

    out_flat = bsa(input.reshape(B * T), filt)
    return out_flat.reshape(B, T)


# speedup vs baseline: 1.7815x; 1.1008x over previous
import functools

import jax
import jax.numpy as jnp
from jax import lax
from jax.experimental import pallas as pl
from jax.experimental.pallas import tpu as pltpu
from jax.experimental.pallas import tpu_sc as plsc

_THRESHOLD = 0.9952
_LANES = 16


def kernel(input, filt):
    B, T = input.shape
    F = filt.shape[0]
    n_steps = T - F
    n_blocks = n_steps // F

    info = plsc.get_sparse_core_info()
    nc, ns = info.num_cores, info.num_subcores
    n_workers = nc * ns
    rows_per_worker = B // n_workers
    chunk = rows_per_worker * T
    nv = F // rows_per_worker

    mesh = plsc.VectorSubcoreMesh(core_axis_name="c", subcore_axis_name="s")

    @functools.partial(
        pl.kernel,
        mesh=mesh,
        compiler_params=pltpu.CompilerParams(needs_layout_passes=False),
        out_type=jax.ShapeDtypeStruct((B * T,), jnp.float32),
        scratch_types=[
            pltpu.VMEM((chunk,), jnp.float32),
            pltpu.VMEM((chunk,), jnp.float32),
            pltpu.VMEM((F,), jnp.float32),
            pltpu.VMEM((F * nv * _LANES,), jnp.float32),
        ],
    )
    def bsa(x_hbm, filt_hbm, out_hbm, x_v, out_v, filt_v, ftab_v):
        wid = lax.axis_index("s") * nc + lax.axis_index("c")
        base = wid * chunk
        pltpu.sync_copy(x_hbm.at[pl.ds(base, chunk)], x_v)
        pltpu.sync_copy(filt_hbm, filt_v)

        lane = lax.iota(jnp.int32, _LANES)
        row4 = (lane >> 2) * T
        pos4 = lane & 3
        thr = jnp.float32(_THRESHOLD)
        one_v = jnp.ones((_LANES,), jnp.float32)
        zero_v = jnp.zeros((_LANES,), jnp.float32)
        posmask = [pos4 == i for i in range(rows_per_worker)]
        xor2 = jax.lax.bitwise_xor(lane, 2)
        xor1 = jax.lax.bitwise_xor(lane, 1)
        bitlane = jnp.int32(1) << lane
        zero_i = jnp.zeros((_LANES,), jnp.int32)
        top_i = zero_i + jnp.int32(1 << (F - 1))
        grp_first = [zero_i + jnp.int32(4 * r) for r in range(rows_per_worker)]

        def shuf(vv, idxv):
            return jnp.take_along_axis(vv, idxv, axis=0,
                                       mode="promise_in_bounds")

        def tree(a):
            p1a = a[0] + a[2]
            p1b = a[1] + a[3]
            p2 = p1a + p1b
            p3 = p2 + shuf(p2, xor2)
            return p3 + shuf(p3, xor1)

        for r in range(F):
            for i in range(nv):
                idx = (pos4 + ((4 * i - r) % F)) & (F - 1)
                ftab_v[pl.ds((r * nv + i) * _LANES, _LANES)] = (
                    plsc.load_gather(filt_v, [idx]))

        def spec(w, taps):
            u = [w[i] - taps[i] for i in range(nv)]
            uu = [u[i] - taps[i] for i in range(nv)]
            a_u = jnp.abs(tree(u))
            a_uu = jnp.abs(tree(uu))
            b_u = a_u * thr
            b_w = jnp.abs(tree(w)) * thr
            return u, uu, a_u, a_uu, b_u, b_w

        w = [plsc.load_gather(x_v, [row4 + pos4 + 4 * i]) for i in range(nv)]
        taps0 = [ftab_v[pl.ds(i * _LANES, _LANES)] for i in range(nv)]
        u, uu, a_u, a_uu, b_u, b_w = spec(w, taps0)
        m0 = jnp.zeros((_LANES,), jnp.bool_)

        def block(jb, carry):
            w = list(carry[:nv])
            u = list(carry[nv:2 * nv])
            uu = list(carry[2 * nv:3 * nv])
            a_u, a_uu, b_u, b_w, m_prev = carry[3 * nv:]
            j0 = jb * F
            gvec0 = row4 + pos4 + (j0 + F)
            g = [plsc.load_gather(x_v, [gvec0 + 4 * i]) for i in range(nv)]
            acc = zero_i
            for k in range(F):
                e1 = jnp.where(m_prev, a_uu, a_u)
                e2 = jnp.where(m_prev, b_u, b_w)
                m = e1 <= e2
                acc = jax.lax.shift_right_logical(acc, 1) | jnp.where(
                    m, top_i, zero_i)
                w = [jnp.where(m_prev, u[i], w[i]) for i in range(nv)]
                ke, le = k // 4, k % 4
                w[ke] = jnp.where(posmask[le], g[ke], w[ke])
                rn = (k + 1) % F
                taps = [ftab_v[pl.ds((rn * nv + i) * _LANES, _LANES)]
                        for i in range(nv)]
                u, uu, a_u, a_uu, b_u, b_w = spec(w, taps)
                m_prev = m
            for r in range(rows_per_worker):
                bits = shuf(acc, grp_first[r]) & bitlane
                spikes = jnp.where(bits != zero_i, one_v, zero_v)
                out_v[pl.ds(r * T + j0, F)] = spikes
            return (*w, *u, *uu, a_u, a_uu, b_u, b_w, m_prev)

        lax.fori_loop(0, n_blocks, block,
                      (*w, *u, *uu, a_u, a_uu, b_u, b_w, m0))

        for r in range(rows_per_worker):
            out_v[pl.ds(r * T + n_steps, F)] = zero_v

        pltpu.sync_copy(out_v, out_hbm.at[pl.ds(base, chunk)])

    out_flat = bsa(input.reshape(B * T), filt)
    return out_flat.reshape(B, T)
